# baseline (speedup 1.0000x reference)
import functools

import jax
import jax.numpy as jnp
from jax import lax
from jax.experimental import pallas as pl
from jax.experimental.pallas import tpu as pltpu
from jax.experimental.pallas import tpu_sc as plsc

_NC = 2
_NS = 16
_CH = 128
_NCH0 = 96
_NCH1 = 64


def _pack_edges(src, dst, n, npad):
    epad = _NS * (_NCH0 + _NCH1) * _CH
    e = src.shape[0]
    pad = epad - e
    src_p = jnp.concatenate([src, jnp.zeros((pad,), jnp.int32)])
    pad_dst = n + jnp.arange(pad, dtype=jnp.int32) % (npad - n)
    dst_p = jnp.concatenate([dst, pad_dst])
    return src_p, dst_p


def _segment_sum_sc(h, src_flat, dst_flat, zeros):
    n, d = h.shape
    ch = _CH
    npad = -(-n // (_NS * 8)) * (_NS * 8)
    rps = npad // _NS

    mesh = plsc.VectorSubcoreMesh(core_axis_name="c", subcore_axis_name="s")

    @functools.partial(
        pl.kernel,
        out_type=jax.ShapeDtypeStruct((_NC, npad, d), jnp.float32),
        mesh=mesh,
        scratch_types=[
            pltpu.VMEM((ch,), jnp.int32),
            pltpu.VMEM((ch,), jnp.int32),
            pltpu.VMEM((ch,), jnp.int32),
            pltpu.VMEM((ch,), jnp.int32),
            pltpu.VMEM((ch, d), jnp.float32),
            pltpu.VMEM((ch, d), jnp.float32),
            pltpu.VMEM_SHARED((npad, d), jnp.float32),
            pltpu.SemaphoreType.DMA,
            pltpu.SemaphoreType.DMA,
            pltpu.SemaphoreType.DMA,
            pltpu.SemaphoreType.DMA,
            pltpu.SemaphoreType.DMA,
            pltpu.SemaphoreType.DMA,
            pltpu.SemaphoreType.DMA,
            pltpu.SemaphoreType.DMA,
        ],
    )
    def seg_kernel(h_hbm, src_hbm, dst_hbm, z_hbm, out_hbm,
                   sidx0, sidx1, didx0, didx1, rows0, rows1, acc,
                   g0, g1, s0, s1, i0, i1, e0, e1):
        c = lax.axis_index("c")
        s = lax.axis_index("s")
        cbase = jnp.where(c == 0, s * _NCH0, _NS * _NCH0 + s * _NCH1)
        half = jnp.where(c == 0, _NCH0 // 2, _NCH1 // 2)

        pltpu.sync_copy(z_hbm, rows0)
        for k in range(rps // ch):
            pltpu.sync_copy(rows0, acc.at[pl.ds(s * rps + k * ch, ch)])
        if rps % ch:
            pltpu.sync_copy(rows0.at[pl.ds(0, rps % ch)],
                            acc.at[pl.ds(s * rps + (rps // ch) * ch, rps % ch)])

        pltpu.async_copy(dst_hbm.at[pl.ds(cbase * ch, ch)], didx0, e0)
        pltpu.async_copy(dst_hbm.at[pl.ds((cbase + 1) * ch, ch)], didx1, e1)
        pltpu.sync_copy(src_hbm.at[pl.ds(cbase * ch, ch)], sidx0)
        pltpu.sync_copy(src_hbm.at[pl.ds((cbase + 1) * ch, ch)], sidx1)
        plsc.subcore_barrier()

        pltpu.sync_copy(h_hbm.at[sidx0], rows0)
        pltpu.make_async_copy(
            dst_hbm.at[pl.ds(cbase * ch, ch)], didx0, e0).wait()
        pltpu.async_copy(rows0, acc.at[didx0], s0, add=True)
        pltpu.async_copy(src_hbm.at[pl.ds((cbase + 2) * ch, ch)], sidx0, i0)
        pltpu.sync_copy(h_hbm.at[sidx1], rows1)
        pltpu.make_async_copy(
            dst_hbm.at[pl.ds(cbase * ch, ch)], didx1, e1).wait()
        pltpu.async_copy(rows1, acc.at[didx1], s1, add=True)
        pltpu.async_copy(src_hbm.at[pl.ds((cbase + 3) * ch, ch)], sidx1, i1)

        @pl.loop(1, half)
        def _(t):
            j0 = cbase + 2 * t
            not_last = t < half - 1

            pltpu.make_async_copy(rows0, acc.at[didx0], s0).wait()
            pltpu.async_copy(dst_hbm.at[pl.ds(j0 * ch, ch)], didx0, e0)
            pltpu.make_async_copy(
                src_hbm.at[pl.ds(cbase * ch, ch)], sidx0, i0).wait()
            pltpu.sync_copy(h_hbm.at[sidx0], rows0)
            pltpu.make_async_copy(
                dst_hbm.at[pl.ds(cbase * ch, ch)], didx0, e0).wait()
            pltpu.async_copy(rows0, acc.at[didx0], s0, add=True)

            @pl.when(not_last)
            def _():
                pltpu.async_copy(
                    src_hbm.at[pl.ds((j0 + 2) * ch, ch)], sidx0, i0)

            pltpu.make_async_copy(rows1, acc.at[didx1], s1).wait()
            pltpu.async_copy(dst_hbm.at[pl.ds((j0 + 1) * ch, ch)], didx1, e1)
            pltpu.make_async_copy(
                src_hbm.at[pl.ds(cbase * ch, ch)], sidx1, i1).wait()
            pltpu.sync_copy(h_hbm.at[sidx1], rows1)
            pltpu.make_async_copy(
                dst_hbm.at[pl.ds(cbase * ch, ch)], didx1, e1).wait()
            pltpu.async_copy(rows1, acc.at[didx1], s1, add=True)

            @pl.when(not_last)
            def _():
                pltpu.async_copy(
                    src_hbm.at[pl.ds((j0 + 3) * ch, ch)], sidx1, i1)

        pltpu.make_async_copy(rows0, acc.at[didx0], s0).wait()
        pltpu.make_async_copy(rows1, acc.at[didx1], s1).wait()
        plsc.subcore_barrier()

        pltpu.sync_copy(acc.at[pl.ds(s * rps, rps)],
                        out_hbm.at[c].at[pl.ds(s * rps, rps)])

    return seg_kernel(h, src_flat, dst_flat, zeros)


def _gin_dense_body(h_ref, p_ref, wa_ref, ba_ref, wb_ref, bb_ref, o_ref):
    z = h_ref[...] + p_ref[0] + p_ref[1]
    z = jnp.dot(z, wa_ref[...], preferred_element_type=jnp.float32,
                precision=lax.Precision.HIGHEST) + ba_ref[...]
    z = jnp.maximum(z, 0.0)
    z = jnp.dot(z, wb_ref[...], preferred_element_type=jnp.float32,
                precision=lax.Precision.HIGHEST) + bb_ref[...]
    o_ref[...] = jnp.maximum(z, 0.0)


def _gin_dense(h, parts, wa, ba, wb, bb, block):
    n, d = h.shape
    grid = (n // block,)
    return pl.pallas_call(
        _gin_dense_body,
        grid=grid,
        in_specs=[
            pl.BlockSpec((block, d), lambda i: (i, 0)),
            pl.BlockSpec((_NC, block, d), lambda i: (0, i, 0)),
            pl.BlockSpec((d, d), lambda i: (0, 0)),
            pl.BlockSpec((1, d), lambda i: (0, 0)),
            pl.BlockSpec((d, d), lambda i: (0, 0)),
            pl.BlockSpec((1, d), lambda i: (0, 0)),
        ],
        out_specs=pl.BlockSpec((block, d), lambda i: (i, 0)),
        out_shape=jax.ShapeDtypeStruct((n, d), jnp.float32),
    )(h, parts, wa, ba, wb, bb)


def _final_body(h_ref, p_ref, wa_ref, ba_ref, wb_ref, bb_ref,
                fw_ref, fb_ref, o_ref):
    z = h_ref[...] + p_ref[0] + p_ref[1]
    z = jnp.dot(z, wa_ref[...], preferred_element_type=jnp.float32,
                precision=lax.Precision.HIGHEST) + ba_ref[...]
    z = jnp.maximum(z, 0.0)
    z = jnp.dot(z, wb_ref[...], preferred_element_type=jnp.float32,
                precision=lax.Precision.HIGHEST) + bb_ref[...]
    z = jnp.maximum(z, 0.0)
    logits = jnp.dot(z, fw_ref[...], preferred_element_type=jnp.float32,
                     precision=lax.Precision.HIGHEST) + fb_ref[...]
    m = jnp.max(logits, axis=1, keepdims=True)
    shifted = logits - m
    lse = jnp.log(jnp.sum(jnp.exp(shifted), axis=1, keepdims=True))
    o_ref[...] = shifted - lse


def _final_layer(h, parts, wa, ba, wb, bb, fw, fb, block):
    n, d = h.shape
    c = fw.shape[1]
    grid = (n // block,)
    return pl.pallas_call(
        _final_body,
        grid=grid,
        in_specs=[
            pl.BlockSpec((block, d), lambda i: (i, 0)),
            pl.BlockSpec((_NC, block, d), lambda i: (0, i, 0)),
            pl.BlockSpec((d, d), lambda i: (0, 0)),
            pl.BlockSpec((1, d), lambda i: (0, 0)),
            pl.BlockSpec((d, d), lambda i: (0, 0)),
            pl.BlockSpec((1, d), lambda i: (0, 0)),
            pl.BlockSpec((d, c), lambda i: (0, 0)),
            pl.BlockSpec((1, c), lambda i: (0, 0)),
        ],
        out_specs=pl.BlockSpec((block, c), lambda i: (i, 0)),
        out_shape=jax.ShapeDtypeStruct((n, c), jnp.float32),
    )(h, parts, wa, ba, wb, bb, fw, fb)


def kernel(x, edge_index, w1a, b1a, w1b, b1b, w2a, b2a, w2b, b2b,
           w3a, b3a, w3b, b3b, fc_w, fc_b):
    src = edge_index[0]
    dst = edge_index[1]
    block = 2000

    b1a_ = b1a.reshape(1, -1)
    b1b_ = b1b.reshape(1, -1)
    b2a_ = b2a.reshape(1, -1)
    b2b_ = b2b.reshape(1, -1)
    b3a_ = b3a.reshape(1, -1)
    b3b_ = b3b.reshape(1, -1)
    fc_b_ = fc_b.reshape(1, -1)

    n = x.shape[0]
    npad = -(-n // (_NS * 8)) * (_NS * 8)
    zeros = jnp.zeros((_CH, x.shape[1]), jnp.float32)
    src3, dst3 = _pack_edges(src, dst, n, npad)

    h = x
    parts = _segment_sum_sc(h, src3, dst3, zeros)
    h = _gin_dense(h, parts, w1a, b1a_, w1b, b1b_, block)
    parts = _segment_sum_sc(h, src3, dst3, zeros)
    h = _gin_dense(h, parts, w2a, b2a_, w2b, b2b_, block)
    parts = _segment_sum_sc(h, src3, dst3, zeros)
    return _final_layer(h, parts, w3a, b3a_, w3b, b3b_, fc_w, fc_b_, block)


# speedup vs baseline: 1.0855x; 1.0855x over previous
import functools

import jax
import jax.numpy as jnp
from jax import lax
from jax.experimental import pallas as pl
from jax.experimental.pallas import tpu as pltpu
from jax.experimental.pallas import tpu_sc as plsc

_NC = 2
_NS = 16
_CH = 128
_NCH0 = 128
_NCH1 = 32


def _pack_edges(src, dst, n, npad):
    epad = _NS * (_NCH0 + _NCH1) * _CH
    e = src.shape[0]
    pad = epad - e
    src_p = jnp.concatenate([src, jnp.zeros((pad,), jnp.int32)])
    pad_dst = n + jnp.arange(pad, dtype=jnp.int32) % (npad - n)
    dst_p = jnp.concatenate([dst, pad_dst])
    return src_p, dst_p


def _segment_sum_sc(h, src_flat, dst_flat, zeros):
    n, d = h.shape
    ch = _CH
    npad = -(-n // (_NS * 8)) * (_NS * 8)
    rps = npad // _NS

    mesh = plsc.VectorSubcoreMesh(core_axis_name="c", subcore_axis_name="s")

    @functools.partial(
        pl.kernel,
        out_type=jax.ShapeDtypeStruct((_NC, npad, d), jnp.float32),
        mesh=mesh,
        scratch_types=[
            pltpu.VMEM((ch,), jnp.int32),
            pltpu.VMEM((ch,), jnp.int32),
            pltpu.VMEM((ch,), jnp.int32),
            pltpu.VMEM((ch,), jnp.int32),
            pltpu.VMEM((ch, d), jnp.float32),
            pltpu.VMEM((ch, d), jnp.float32),
            pltpu.VMEM_SHARED((npad, d), jnp.float32),
            pltpu.SemaphoreType.DMA,
            pltpu.SemaphoreType.DMA,
            pltpu.SemaphoreType.DMA,
            pltpu.SemaphoreType.DMA,
            pltpu.SemaphoreType.DMA,
            pltpu.SemaphoreType.DMA,
            pltpu.SemaphoreType.DMA,
            pltpu.SemaphoreType.DMA,
        ],
    )
    def seg_kernel(h_hbm, src_hbm, dst_hbm, z_hbm, out_hbm,
                   sidx0, sidx1, didx0, didx1, rows0, rows1, acc,
                   g0, g1, s0, s1, i0, i1, e0, e1):
        c = lax.axis_index("c")
        s = lax.axis_index("s")
        cbase = jnp.where(c == 0, s * _NCH0, _NS * _NCH0 + s * _NCH1)
        half = jnp.where(c == 0, _NCH0 // 2, _NCH1 // 2)

        pltpu.sync_copy(z_hbm, rows0)
        for k in range(rps // ch):
            pltpu.sync_copy(rows0, acc.at[pl.ds(s * rps + k * ch, ch)])
        if rps % ch:
            pltpu.sync_copy(rows0.at[pl.ds(0, rps % ch)],
                            acc.at[pl.ds(s * rps + (rps // ch) * ch, rps % ch)])

        pltpu.async_copy(dst_hbm.at[pl.ds(cbase * ch, ch)], didx0, e0)
        pltpu.async_copy(dst_hbm.at[pl.ds((cbase + 1) * ch, ch)], didx1, e1)
        pltpu.sync_copy(src_hbm.at[pl.ds(cbase * ch, ch)], sidx0)
        pltpu.sync_copy(src_hbm.at[pl.ds((cbase + 1) * ch, ch)], sidx1)
        plsc.subcore_barrier()

        pltpu.sync_copy(h_hbm.at[sidx0], rows0)
        pltpu.make_async_copy(
            dst_hbm.at[pl.ds(cbase * ch, ch)], didx0, e0).wait()
        pltpu.async_copy(rows0, acc.at[didx0], s0, add=True)
        pltpu.async_copy(src_hbm.at[pl.ds((cbase + 2) * ch, ch)], sidx0, i0)
        pltpu.sync_copy(h_hbm.at[sidx1], rows1)
        pltpu.make_async_copy(
            dst_hbm.at[pl.ds(cbase * ch, ch)], didx1, e1).wait()
        pltpu.async_copy(rows1, acc.at[didx1], s1, add=True)
        pltpu.async_copy(src_hbm.at[pl.ds((cbase + 3) * ch, ch)], sidx1, i1)

        @pl.loop(1, half)
        def _(t):
            j0 = cbase + 2 * t
            not_last = t < half - 1

            pltpu.make_async_copy(rows0, acc.at[didx0], s0).wait()
            pltpu.async_copy(dst_hbm.at[pl.ds(j0 * ch, ch)], didx0, e0)
            pltpu.make_async_copy(
                src_hbm.at[pl.ds(cbase * ch, ch)], sidx0, i0).wait()
            pltpu.sync_copy(h_hbm.at[sidx0], rows0)
            pltpu.make_async_copy(
                dst_hbm.at[pl.ds(cbase * ch, ch)], didx0, e0).wait()
            pltpu.async_copy(rows0, acc.at[didx0], s0, add=True)

            @pl.when(not_last)
            def _():
                pltpu.async_copy(
                    src_hbm.at[pl.ds((j0 + 2) * ch, ch)], sidx0, i0)

            pltpu.make_async_copy(rows1, acc.at[didx1], s1).wait()
            pltpu.async_copy(dst_hbm.at[pl.ds((j0 + 1) * ch, ch)], didx1, e1)
            pltpu.make_async_copy(
                src_hbm.at[pl.ds(cbase * ch, ch)], sidx1, i1).wait()
            pltpu.sync_copy(h_hbm.at[sidx1], rows1)
            pltpu.make_async_copy(
                dst_hbm.at[pl.ds(cbase * ch, ch)], didx1, e1).wait()
            pltpu.async_copy(rows1, acc.at[didx1], s1, add=True)

            @pl.when(not_last)
            def _():
                pltpu.async_copy(
                    src_hbm.at[pl.ds((j0 + 3) * ch, ch)], sidx1, i1)

        pltpu.make_async_copy(rows0, acc.at[didx0], s0).wait()
        pltpu.make_async_copy(rows1, acc.at[didx1], s1).wait()
        plsc.subcore_barrier()

        pltpu.sync_copy(acc.at[pl.ds(s * rps, rps)],
                        out_hbm.at[c].at[pl.ds(s * rps, rps)])

    return seg_kernel(h, src_flat, dst_flat, zeros)


def _gin_dense_body(h_ref, p_ref, wa_ref, ba_ref, wb_ref, bb_ref, o_ref):
    z = h_ref[...] + p_ref[0] + p_ref[1]
    z = jnp.dot(z, wa_ref[...], preferred_element_type=jnp.float32,
                precision=lax.Precision.HIGHEST) + ba_ref[...]
    z = jnp.maximum(z, 0.0)
    z = jnp.dot(z, wb_ref[...], preferred_element_type=jnp.float32,
                precision=lax.Precision.HIGHEST) + bb_ref[...]
    o_ref[...] = jnp.maximum(z, 0.0)


def _gin_dense(h, parts, wa, ba, wb, bb, block):
    n, d = h.shape
    grid = (n // block,)
    return pl.pallas_call(
        _gin_dense_body,
        grid=grid,
        in_specs=[
            pl.BlockSpec((block, d), lambda i: (i, 0)),
            pl.BlockSpec((_NC, block, d), lambda i: (0, i, 0)),
            pl.BlockSpec((d, d), lambda i: (0, 0)),
            pl.BlockSpec((1, d), lambda i: (0, 0)),
            pl.BlockSpec((d, d), lambda i: (0, 0)),
            pl.BlockSpec((1, d), lambda i: (0, 0)),
        ],
        out_specs=pl.BlockSpec((block, d), lambda i: (i, 0)),
        out_shape=jax.ShapeDtypeStruct((n, d), jnp.float32),
    )(h, parts, wa, ba, wb, bb)


def _final_body(h_ref, p_ref, wa_ref, ba_ref, wb_ref, bb_ref,
                fw_ref, fb_ref, o_ref):
    z = h_ref[...] + p_ref[0] + p_ref[1]
    z = jnp.dot(z, wa_ref[...], preferred_element_type=jnp.float32,
                precision=lax.Precision.HIGHEST) + ba_ref[...]
    z = jnp.maximum(z, 0.0)
    z = jnp.dot(z, wb_ref[...], preferred_element_type=jnp.float32,
                precision=lax.Precision.HIGHEST) + bb_ref[...]
    z = jnp.maximum(z, 0.0)
    logits = jnp.dot(z, fw_ref[...], preferred_element_type=jnp.float32,
                     precision=lax.Precision.HIGHEST) + fb_ref[...]
    m = jnp.max(logits, axis=1, keepdims=True)
    shifted = logits - m
    lse = jnp.log(jnp.sum(jnp.exp(shifted), axis=1, keepdims=True))
    o_ref[...] = shifted - lse


def _final_layer(h, parts, wa, ba, wb, bb, fw, fb, block):
    n, d = h.shape
    c = fw.shape[1]
    grid = (n // block,)
    return pl.pallas_call(
        _final_body,
        grid=grid,
        in_specs=[
            pl.BlockSpec((block, d), lambda i: (i, 0)),
            pl.BlockSpec((_NC, block, d), lambda i: (0, i, 0)),
            pl.BlockSpec((d, d), lambda i: (0, 0)),
            pl.BlockSpec((1, d), lambda i: (0, 0)),
            pl.BlockSpec((d, d), lambda i: (0, 0)),
            pl.BlockSpec((1, d), lambda i: (0, 0)),
            pl.BlockSpec((d, c), lambda i: (0, 0)),
            pl.BlockSpec((1, c), lambda i: (0, 0)),
        ],
        out_specs=pl.BlockSpec((block, c), lambda i: (i, 0)),
        out_shape=jax.ShapeDtypeStruct((n, c), jnp.float32),
    )(h, parts, wa, ba, wb, bb, fw, fb)


def kernel(x, edge_index, w1a, b1a, w1b, b1b, w2a, b2a, w2b, b2b,
           w3a, b3a, w3b, b3b, fc_w, fc_b):
    src = edge_index[0]
    dst = edge_index[1]
    block = 2000

    b1a_ = b1a.reshape(1, -1)
    b1b_ = b1b.reshape(1, -1)
    b2a_ = b2a.reshape(1, -1)
    b2b_ = b2b.reshape(1, -1)
    b3a_ = b3a.reshape(1, -1)
    b3b_ = b3b.reshape(1, -1)
    fc_b_ = fc_b.reshape(1, -1)

    n = x.shape[0]
    npad = -(-n // (_NS * 8)) * (_NS * 8)
    zeros = jnp.zeros((_CH, x.shape[1]), jnp.float32)
    src3, dst3 = _pack_edges(src, dst, n, npad)

    h = x
    parts = _segment_sum_sc(h, src3, dst3, zeros)
    h = _gin_dense(h, parts, w1a, b1a_, w1b, b1b_, block)
    parts = _segment_sum_sc(h, src3, dst3, zeros)
    h = _gin_dense(h, parts, w2a, b2a_, w2b, b2b_, block)
    parts = _segment_sum_sc(h, src3, dst3, zeros)
    return _final_layer(h, parts, w3a, b3a_, w3b, b3b_, fc_w, fc_b_, block)


# speedup vs baseline: 1.4865x; 1.3695x over previous
import functools

import jax
import jax.numpy as jnp
from jax import lax
from jax.experimental import pallas as pl
from jax.experimental.pallas import tpu as pltpu
from jax.experimental.pallas import tpu_sc as plsc

_NC = 2
_NS = 16
_CH = 80


def _segment_sum_sc(h, src, dst, zeros):
    n, d = h.shape
    e = src.shape[0]
    epw = e // (_NC * _NS)
    nchunk = epw // _CH
    npad = -(-n // (_NS * 8)) * (_NS * 8)
    rows_per_sub = npad // _NS

    mesh = plsc.VectorSubcoreMesh(core_axis_name="c", subcore_axis_name="s")

    @functools.partial(
        pl.kernel,
        out_type=jax.ShapeDtypeStruct((_NC, npad, d), jnp.float32),
        mesh=mesh,
        scratch_types=[
            pltpu.VMEM((_CH,), jnp.int32),
            pltpu.VMEM((_CH,), jnp.int32),
            pltpu.VMEM((_CH, d), jnp.float32),
            pltpu.VMEM_SHARED((npad, d), jnp.float32),
        ],
    )
    def seg_kernel(h_hbm, src_hbm, dst_hbm, z_hbm, out_hbm, sidx, didx, rows, acc):
        c = lax.axis_index("c")
        s = lax.axis_index("s")

        pltpu.sync_copy(
            z_hbm.at[pl.ds(s * rows_per_sub, rows_per_sub)],
            acc.at[pl.ds(s * rows_per_sub, rows_per_sub)],
        )
        plsc.subcore_barrier()

        base = (c * _NS + s) * epw

        @pl.loop(0, nchunk)
        def _(i):
            off = base + i * _CH
            pltpu.sync_copy(src_hbm.at[pl.ds(off, _CH)], sidx)
            pltpu.sync_copy(dst_hbm.at[pl.ds(off, _CH)], didx)
            pltpu.sync_copy(h_hbm.at[sidx], rows)
            pltpu.sync_copy(rows, acc.at[didx], add=True)

        plsc.subcore_barrier()

        pltpu.sync_copy(
            acc.at[pl.ds(s * rows_per_sub, rows_per_sub)],
            out_hbm.at[c].at[pl.ds(s * rows_per_sub, rows_per_sub)],
        )

    return seg_kernel(h, src, dst, zeros)


def _gin_dense_body(h_ref, p_ref, wa_ref, ba_ref, wb_ref, bb_ref, o_ref):
    z = h_ref[...] + p_ref[0] + p_ref[1]
    z = jnp.dot(z, wa_ref[...], preferred_element_type=jnp.float32,
                precision=lax.Precision.HIGHEST) + ba_ref[...]
    z = jnp.maximum(z, 0.0)
    z = jnp.dot(z, wb_ref[...], preferred_element_type=jnp.float32,
                precision=lax.Precision.HIGHEST) + bb_ref[...]
    o_ref[...] = jnp.maximum(z, 0.0)


def _gin_dense(h, parts, wa, ba, wb, bb, block):
    n, d = h.shape
    grid = (n // block,)
    return pl.pallas_call(
        _gin_dense_body,
        grid=grid,
        in_specs=[
            pl.BlockSpec((block, d), lambda i: (i, 0)),
            pl.BlockSpec((_NC, block, d), lambda i: (0, i, 0)),
            pl.BlockSpec((d, d), lambda i: (0, 0)),
            pl.BlockSpec((1, d), lambda i: (0, 0)),
            pl.BlockSpec((d, d), lambda i: (0, 0)),
            pl.BlockSpec((1, d), lambda i: (0, 0)),
        ],
        out_specs=pl.BlockSpec((block, d), lambda i: (i, 0)),
        out_shape=jax.ShapeDtypeStruct((n, d), jnp.float32),
    )(h, parts, wa, ba, wb, bb)


def _final_body(h_ref, p_ref, wa_ref, ba_ref, wb_ref, bb_ref,
                fw_ref, fb_ref, o_ref):
    z = h_ref[...] + p_ref[0] + p_ref[1]
    z = jnp.dot(z, wa_ref[...], preferred_element_type=jnp.float32,
                precision=lax.Precision.HIGHEST) + ba_ref[...]
    z = jnp.maximum(z, 0.0)
    z = jnp.dot(z, wb_ref[...], preferred_element_type=jnp.float32,
                precision=lax.Precision.HIGHEST) + bb_ref[...]
    z = jnp.maximum(z, 0.0)
    logits = jnp.dot(z, fw_ref[...], preferred_element_type=jnp.float32,
                     precision=lax.Precision.HIGHEST) + fb_ref[...]
    m = jnp.max(logits, axis=1, keepdims=True)
    shifted = logits - m
    lse = jnp.log(jnp.sum(jnp.exp(shifted), axis=1, keepdims=True))
    o_ref[...] = shifted - lse


def _final_layer(h, parts, wa, ba, wb, bb, fw, fb, block):
    n, d = h.shape
    c = fw.shape[1]
    grid = (n // block,)
    return pl.pallas_call(
        _final_body,
        grid=grid,
        in_specs=[
            pl.BlockSpec((block, d), lambda i: (i, 0)),
            pl.BlockSpec((_NC, block, d), lambda i: (0, i, 0)),
            pl.BlockSpec((d, d), lambda i: (0, 0)),
            pl.BlockSpec((1, d), lambda i: (0, 0)),
            pl.BlockSpec((d, d), lambda i: (0, 0)),
            pl.BlockSpec((1, d), lambda i: (0, 0)),
            pl.BlockSpec((d, c), lambda i: (0, 0)),
            pl.BlockSpec((1, c), lambda i: (0, 0)),
        ],
        out_specs=pl.BlockSpec((block, c), lambda i: (i, 0)),
        out_shape=jax.ShapeDtypeStruct((n, c), jnp.float32),
    )(h, parts, wa, ba, wb, bb, fw, fb)


def kernel(x, edge_index, w1a, b1a, w1b, b1b, w2a, b2a, w2b, b2b,
           w3a, b3a, w3b, b3b, fc_w, fc_b):
    src = edge_index[0]
    dst = edge_index[1]
    block = 2000

    b1a_ = b1a.reshape(1, -1)
    b1b_ = b1b.reshape(1, -1)
    b2a_ = b2a.reshape(1, -1)
    b2b_ = b2b.reshape(1, -1)
    b3a_ = b3a.reshape(1, -1)
    b3b_ = b3b.reshape(1, -1)
    fc_b_ = fc_b.reshape(1, -1)

    n = x.shape[0]
    npad = -(-n // (_NS * 8)) * (_NS * 8)
    zeros = jnp.zeros((npad, x.shape[1]), jnp.float32)

    h = x
    parts = _segment_sum_sc(h, src, dst, zeros)
    h = _gin_dense(h, parts, w1a, b1a_, w1b, b1b_, block)
    parts = _segment_sum_sc(h, src, dst, zeros)
    h = _gin_dense(h, parts, w2a, b2a_, w2b, b2b_, block)
    parts = _segment_sum_sc(h, src, dst, zeros)
    return _final_layer(h, parts, w3a, b3a_, w3b, b3b_, fc_w, fc_b_, block)


# speedup vs baseline: 1.6388x; 1.1024x over previous
import functools

import jax
import jax.numpy as jnp
from jax import lax
from jax.experimental import pallas as pl
from jax.experimental.pallas import tpu as pltpu
from jax.experimental.pallas import tpu_sc as plsc

_NC = 2
_NS = 16
_CH = 80


def _pack_edges(src, dst):
    e = src.shape[0]
    nch = e // _CH
    return jnp.concatenate(
        [src.reshape(nch, 1, _CH), dst.reshape(nch, 1, _CH),
         jnp.zeros((nch, 6, _CH), jnp.int32)], axis=1)


def _segment_sum_sc(h, pk, zeros):
    n, d = h.shape
    e = pk.shape[0] * _CH
    epw = e // (_NC * _NS)
    nchunk = epw // _CH
    npad = -(-n // (_NS * 8)) * (_NS * 8)
    rows_per_sub = npad // _NS

    mesh = plsc.VectorSubcoreMesh(core_axis_name="c", subcore_axis_name="s")

    @functools.partial(
        pl.kernel,
        out_type=jax.ShapeDtypeStruct((_NC, npad, d), jnp.float32),
        mesh=mesh,
        scratch_types=[
            pltpu.VMEM((8, _CH), jnp.int32),
            pltpu.VMEM((_CH, d), jnp.float32),
            pltpu.VMEM_SHARED((npad, d), jnp.float32),
        ],
    )
    def seg_kernel(h_hbm, pk_hbm, z_hbm, out_hbm, ibuf, rows, acc):
        c = lax.axis_index("c")
        s = lax.axis_index("s")

        pltpu.sync_copy(
            z_hbm.at[pl.ds(s * rows_per_sub, rows_per_sub)],
            acc.at[pl.ds(s * rows_per_sub, rows_per_sub)],
        )
        plsc.subcore_barrier()

        cbase = (c * _NS + s) * nchunk

        @pl.loop(0, nchunk)
        def _(i):
            pltpu.sync_copy(pk_hbm.at[cbase + i], ibuf)
            pltpu.sync_copy(h_hbm.at[ibuf.at[0]], rows)
            pltpu.sync_copy(rows, acc.at[ibuf.at[1]], add=True)

        plsc.subcore_barrier()

        pltpu.sync_copy(
            acc.at[pl.ds(s * rows_per_sub, rows_per_sub)],
            out_hbm.at[c].at[pl.ds(s * rows_per_sub, rows_per_sub)],
        )

    return seg_kernel(h, pk, zeros)


def _gin_dense_body(h_ref, p_ref, wa_ref, ba_ref, wb_ref, bb_ref, o_ref):
    z = h_ref[...] + p_ref[0] + p_ref[1]
    z = jnp.dot(z, wa_ref[...], preferred_element_type=jnp.float32,
                precision=lax.Precision.HIGHEST) + ba_ref[...]
    z = jnp.maximum(z, 0.0)
    z = jnp.dot(z, wb_ref[...], preferred_element_type=jnp.float32,
                precision=lax.Precision.HIGHEST) + bb_ref[...]
    o_ref[...] = jnp.maximum(z, 0.0)


def _gin_dense(h, parts, wa, ba, wb, bb, block):
    n, d = h.shape
    grid = (n // block,)
    return pl.pallas_call(
        _gin_dense_body,
        grid=grid,
        in_specs=[
            pl.BlockSpec((block, d), lambda i: (i, 0)),
            pl.BlockSpec((_NC, block, d), lambda i: (0, i, 0)),
            pl.BlockSpec((d, d), lambda i: (0, 0)),
            pl.BlockSpec((1, d), lambda i: (0, 0)),
            pl.BlockSpec((d, d), lambda i: (0, 0)),
            pl.BlockSpec((1, d), lambda i: (0, 0)),
        ],
        out_specs=pl.BlockSpec((block, d), lambda i: (i, 0)),
        out_shape=jax.ShapeDtypeStruct((n, d), jnp.float32),
    )(h, parts, wa, ba, wb, bb)


def _final_body(h_ref, p_ref, wa_ref, ba_ref, wb_ref, bb_ref,
                fw_ref, fb_ref, o_ref):
    z = h_ref[...] + p_ref[0] + p_ref[1]
    z = jnp.dot(z, wa_ref[...], preferred_element_type=jnp.float32,
                precision=lax.Precision.HIGHEST) + ba_ref[...]
    z = jnp.maximum(z, 0.0)
    z = jnp.dot(z, wb_ref[...], preferred_element_type=jnp.float32,
                precision=lax.Precision.HIGHEST) + bb_ref[...]
    z = jnp.maximum(z, 0.0)
    logits = jnp.dot(z, fw_ref[...], preferred_element_type=jnp.float32,
                     precision=lax.Precision.HIGHEST) + fb_ref[...]
    m = jnp.max(logits, axis=1, keepdims=True)
    shifted = logits - m
    lse = jnp.log(jnp.sum(jnp.exp(shifted), axis=1, keepdims=True))
    o_ref[...] = shifted - lse


def _final_layer(h, parts, wa, ba, wb, bb, fw, fb, block):
    n, d = h.shape
    c = fw.shape[1]
    grid = (n // block,)
    return pl.pallas_call(
        _final_body,
        grid=grid,
        in_specs=[
            pl.BlockSpec((block, d), lambda i: (i, 0)),
            pl.BlockSpec((_NC, block, d), lambda i: (0, i, 0)),
            pl.BlockSpec((d, d), lambda i: (0, 0)),
            pl.BlockSpec((1, d), lambda i: (0, 0)),
            pl.BlockSpec((d, d), lambda i: (0, 0)),
            pl.BlockSpec((1, d), lambda i: (0, 0)),
            pl.BlockSpec((d, c), lambda i: (0, 0)),
            pl.BlockSpec((1, c), lambda i: (0, 0)),
        ],
        out_specs=pl.BlockSpec((block, c), lambda i: (i, 0)),
        out_shape=jax.ShapeDtypeStruct((n, c), jnp.float32),
    )(h, parts, wa, ba, wb, bb, fw, fb)


def kernel(x, edge_index, w1a, b1a, w1b, b1b, w2a, b2a, w2b, b2b,
           w3a, b3a, w3b, b3b, fc_w, fc_b):
    src = edge_index[0]
    dst = edge_index[1]
    block = 2000

    b1a_ = b1a.reshape(1, -1)
    b1b_ = b1b.reshape(1, -1)
    b2a_ = b2a.reshape(1, -1)
    b2b_ = b2b.reshape(1, -1)
    b3a_ = b3a.reshape(1, -1)
    b3b_ = b3b.reshape(1, -1)
    fc_b_ = fc_b.reshape(1, -1)

    n = x.shape[0]
    npad = -(-n // (_NS * 8)) * (_NS * 8)
    zeros = jnp.zeros((npad, x.shape[1]), jnp.float32)
    pk = _pack_edges(src, dst)

    h = x
    parts = _segment_sum_sc(h, pk, zeros)
    h = _gin_dense(h, parts, w1a, b1a_, w1b, b1b_, block)
    parts = _segment_sum_sc(h, pk, zeros)
    h = _gin_dense(h, parts, w2a, b2a_, w2b, b2b_, block)
    parts = _segment_sum_sc(h, pk, zeros)
    return _final_layer(h, parts, w3a, b3a_, w3b, b3b_, fc_w, fc_b_, block)
